# baseline (speedup 1.0000x reference)
import jax
import jax.numpy as jnp
from jax.experimental import pallas as pl
from jax.experimental.pallas import tpu as pltpu

GRID_X, GRID_Y = 256, 256
NUM_CELLS = GRID_X * GRID_Y
FEAT = 64
VOXEL = jnp.array([0.2, 0.2, 6.0], dtype=jnp.float32)
PC_LO = jnp.array([-25.6, -25.6, -3.0], dtype=jnp.float32)


def _pillar_sums(points, W, b):
    coords = jnp.floor((points - PC_LO) / VOXEL).astype(jnp.int32)
    coords = jnp.clip(coords, 0, jnp.array([GRID_X - 1, GRID_Y - 1, 0], dtype=jnp.int32))
    flat = coords[:, 1] * GRID_X + coords[:, 0]
    ones = jnp.ones((points.shape[0],), dtype=jnp.float32)
    counts = jax.ops.segment_sum(ones, flat, num_segments=NUM_CELLS)
    denom = jnp.maximum(counts, 1.0)
    sum_xyz = jax.ops.segment_sum(points, flat, num_segments=NUM_CELLS)
    mean_xyz = sum_xyz / denom[:, None]
    f_cluster = points - mean_xyz[flat]
    vcenter = (coords.astype(jnp.float32) + 0.5) * VOXEL + PC_LO
    f_center = points - vcenter
    feats_in = jnp.concatenate([points, f_cluster, f_center], axis=1)
    point_feats = jax.nn.relu(feats_in @ W + b)
    vox_sum = jax.ops.segment_sum(point_feats, flat, num_segments=NUM_CELLS)
    return vox_sum, denom, point_feats


def _combine_body(s1_ref, s0_ref, sh_ref, d1_ref, d0_ref, dh_ref, feat_ref, v1_ref):
    v1 = s1_ref[...] / d1_ref[...]
    v0 = s0_ref[...] / d0_ref[...]
    vh = sh_ref[...] / dh_ref[...]
    v1_ref[...] = v1
    feat_ref[...] = v1 - 0.5 * (v0 + vh)


def kernel(pc1s, pc0s, pch1s, W, b):
    B = pc1s.shape[0]
    pillar = jax.vmap(lambda p: _pillar_sums(p, W, b))
    s1, d1, _ = pillar(pc1s)
    s0, d0, p0_feats = pillar(pc0s)
    sh, dh, _ = pillar(pch1s)

    ROWS = 2048
    grid = (B, NUM_CELLS // ROWS)
    sum_spec = pl.BlockSpec((1, ROWS, FEAT), lambda i, j: (i, j, 0))
    den_spec = pl.BlockSpec((1, ROWS, 1), lambda i, j: (i, j, 0))
    features, v1 = pl.pallas_call(
        _combine_body,
        grid=grid,
        in_specs=[sum_spec, sum_spec, sum_spec, den_spec, den_spec, den_spec],
        out_specs=[sum_spec, sum_spec],
        out_shape=[
            jax.ShapeDtypeStruct((B, NUM_CELLS, FEAT), jnp.float32),
            jax.ShapeDtypeStruct((B, NUM_CELLS, FEAT), jnp.float32),
        ],
    )(s1, s0, sh, d1[..., None], d0[..., None], dh[..., None])
    return features, v1, p0_feats


# speedup vs baseline: 1.5768x; 1.5768x over previous
import functools

import jax
import jax.numpy as jnp
from jax import lax
from jax.experimental import pallas as pl
from jax.experimental.pallas import tpu as pltpu
from jax.experimental.pallas import tpu_sc as plsc

GRID_X, GRID_Y = 256, 256
NUM_CELLS = GRID_X * GRID_Y
FEAT = 64
N_REAL = 100000
NPAD = 102400
PTS_PER_TILE = NPAD // 32
NCELL_PAD = 66048
TRASH = NUM_CELLS
NSETS = 6
CH = 2048

VX = 0.2
VY = 0.2
LOX = -25.6
LOY = -25.6

_MESH = plsc.VectorSubcoreMesh(core_axis_name="c", subcore_axis_name="s")
_SC_PARAMS = pltpu.CompilerParams(needs_layout_passes=False)


def _pillar_ids(xv, yv, gidx):
    cx = ((xv - LOX) / VX).astype(jnp.int32)
    cy = ((yv - LOY) / VY).astype(jnp.int32)
    cx = jnp.minimum(jnp.maximum(cx, 0), GRID_X - 1)
    cy = jnp.minimum(jnp.maximum(cy, 0), GRID_Y - 1)
    pid = cy * GRID_X + cx
    return jnp.where(gidx >= N_REAL, TRASH, pid)


def _k1_body(pts_hbm, ids_hbm, gridat_hbm, grid_v, xb, yb, sb, idb):
    c = lax.axis_index("c")
    s = lax.axis_index("s")
    wid = s * 2 + c
    cb = wid // 4
    col = wid % 4
    zero16 = jnp.zeros((16,), jnp.float32)
    iota = lax.iota(jnp.int32, 16)

    def zero_grid(i, _):
        off = pl.multiple_of(i * 16, 16)
        grid_v[pl.ds(off, 16)] = zero16
        return _

    def chunk_ids(ch, store_ids):
        base = ch * CH
        pltpu.sync_copy(pts_hbm.at[pl.ds((cb * 3 + 0) * NPAD + base, CH)], xb)
        pltpu.sync_copy(pts_hbm.at[pl.ds((cb * 3 + 1) * NPAD + base, CH)], yb)

        def body(j, _):
            off = pl.multiple_of(j * 16, 16)
            gidx = base + off + iota
            pid = _pillar_ids(xb[pl.ds(off, 16)], yb[pl.ds(off, 16)], gidx)
            if store_ids:
                idb[pl.ds(off, 16)] = pid
                plsc.addupdate_scatter(grid_v, [pid], jnp.ones((16,), jnp.float32))
            else:
                plsc.addupdate_scatter(grid_v, [pid], sb[pl.ds(off, 16)])
            return _

        return body

    @pl.when(jnp.logical_and(wid < 24, col == 0))
    def _():
        lax.fori_loop(0, NCELL_PAD // 16, zero_grid, None)

        def run(ch, _):
            lax.fori_loop(0, CH // 16, chunk_ids(ch, True), None)
            pltpu.sync_copy(idb, ids_hbm.at[pl.ds(cb * NPAD + ch * CH, CH)])
            return _

        lax.fori_loop(0, NPAD // CH, run, None)
        pltpu.sync_copy(grid_v, gridat_hbm.at[pl.ds(cb * 4 * NCELL_PAD, NCELL_PAD)])

    @pl.when(jnp.logical_and(wid < 24, col > 0))
    def _():
        lax.fori_loop(0, NCELL_PAD // 16, zero_grid, None)

        def run(ch, _):
            pltpu.sync_copy(
                pts_hbm.at[pl.ds((cb * 3 + col - 1) * NPAD + ch * CH, CH)], sb)
            lax.fori_loop(0, CH // 16, chunk_ids(ch, False), None)
            return _

        lax.fori_loop(0, NPAD // CH, run, None)
        pltpu.sync_copy(grid_v,
                        gridat_hbm.at[pl.ds((cb * 4 + col) * NCELL_PAD, NCELL_PAD)])


def _k1(pts):
    return pl.kernel(
        _k1_body,
        out_type=[
            jax.ShapeDtypeStruct((NSETS * NPAD,), jnp.int32),
            jax.ShapeDtypeStruct((NSETS * 4 * NCELL_PAD,), jnp.float32),
        ],
        mesh=_MESH,
        scratch_types=[
            pltpu.VMEM((NCELL_PAD,), jnp.float32),
            pltpu.VMEM((CH,), jnp.float32),
            pltpu.VMEM((CH,), jnp.float32),
            pltpu.VMEM((CH,), jnp.float32),
            pltpu.VMEM((CH,), jnp.int32),
        ],
        compiler_params=_SC_PARAMS,
    )(pts)


def _k15_body(ga_ref, mean_ref, invd_ref):
    cnt = ga_ref[0, 0, :]
    den = jnp.maximum(cnt, 1.0)
    mean_ref[0] = ga_ref[0, 1:4, :] / den[None, :]
    invd_ref[0, 0] = 1.0 / den


def _k15(gridat):
    return pl.pallas_call(
        _k15_body,
        grid=(NSETS,),
        in_specs=[pl.BlockSpec((1, 4, NCELL_PAD), lambda i: (i, 0, 0))],
        out_specs=[
            pl.BlockSpec((1, 3, NCELL_PAD), lambda i: (i, 0, 0)),
            pl.BlockSpec((1, 1, NCELL_PAD), lambda i: (i, 0, 0)),
        ],
        out_shape=[
            jax.ShapeDtypeStruct((NSETS, 3, NCELL_PAD), jnp.float32),
            jax.ShapeDtypeStruct((NSETS, 1, NCELL_PAD), jnp.float32),
        ],
    )(gridat)


def _k2_body(pts_hbm, mean_hbm, fcl_hbm, colb, xb, yb, zb, idb, fb):
    c = lax.axis_index("c")
    s = lax.axis_index("s")
    wid = s * 2 + c
    base = wid * PTS_PER_TILE
    iota = lax.iota(jnp.int32, 16)

    for cb in range(NSETS):
        pltpu.sync_copy(pts_hbm.at[pl.ds((cb * 3 + 0) * NPAD + base, PTS_PER_TILE)], xb)
        pltpu.sync_copy(pts_hbm.at[pl.ds((cb * 3 + 1) * NPAD + base, PTS_PER_TILE)], yb)
        pltpu.sync_copy(pts_hbm.at[pl.ds((cb * 3 + 2) * NPAD + base, PTS_PER_TILE)], zb)

        def mkids(j, _):
            off = pl.multiple_of(j * 16, 16)
            gidx = base + off + iota
            idb[pl.ds(off, 16)] = _pillar_ids(xb[pl.ds(off, 16)],
                                              yb[pl.ds(off, 16)], gidx)
            return _

        lax.fori_loop(0, PTS_PER_TILE // 16, mkids, None)

        for p, pbuf in enumerate((xb, yb, zb)):
            pltpu.sync_copy(
                mean_hbm.at[pl.ds((cb * 3 + p) * NCELL_PAD, NCELL_PAD)], colb)

            def gat(j, _, pbuf=pbuf):
                off = pl.multiple_of(j * 16, 16)
                pid = idb[pl.ds(off, 16)]
                m = plsc.load_gather(colb, [pid])
                fb[pl.ds(off, 16)] = pbuf[pl.ds(off, 16)] - m
                return _

            lax.fori_loop(0, PTS_PER_TILE // 16, gat, None)
            pltpu.sync_copy(
                fb, fcl_hbm.at[pl.ds((cb * 3 + p) * NPAD + base, PTS_PER_TILE)])


def _k2(pts, meant):
    return pl.kernel(
        _k2_body,
        out_type=jax.ShapeDtypeStruct((NSETS * 3 * NPAD,), jnp.float32),
        mesh=_MESH,
        scratch_types=[
            pltpu.VMEM((NCELL_PAD,), jnp.float32),
            pltpu.VMEM((PTS_PER_TILE,), jnp.float32),
            pltpu.VMEM((PTS_PER_TILE,), jnp.float32),
            pltpu.VMEM((PTS_PER_TILE,), jnp.float32),
            pltpu.VMEM((PTS_PER_TILE,), jnp.int32),
            pltpu.VMEM((PTS_PER_TILE,), jnp.float32),
        ],
        compiler_params=_SC_PARAMS,
    )(pts, meant)


def _featmul(pts_blk, fcl_blk, ids_blk, w_blk, b_blk):
    fid = ids_blk[0, 0]
    cx = (fid & (GRID_X - 1)).astype(jnp.float32)
    cy = (fid >> 8).astype(jnp.float32)
    vcx = (cx + 0.5) * VX + LOX
    vcy = (cy + 0.5) * VY + LOY
    fcx = pts_blk[0, 0, :] - vcx
    fcy = pts_blk[0, 1, :] - vcy
    fcz = pts_blk[0, 2, :]
    fcs = jnp.stack([fcx, fcy, fcz], axis=0)
    lhs = jnp.concatenate([pts_blk[0], fcl_blk[0], fcs], axis=0)
    out = lax.dot_general(lhs, w_blk,
                          dimension_numbers=(((0,), (0,)), ((), ())),
                          preferred_element_type=jnp.float32)
    return jax.nn.relu(out + b_blk[None, :])


def _k25_body(pts_ref, fcl_ref, ids_ref, w_ref, b_ref, pft_ref):
    out = _featmul(pts_ref[...], fcl_ref[...], ids_ref[...], w_ref[...], b_ref[...])
    pft_ref[0] = out.T


def _k25(pts, fcl, ids, W, b):
    blk = 512
    return pl.pallas_call(
        _k25_body,
        grid=(NSETS, NPAD // blk),
        in_specs=[
            pl.BlockSpec((1, 3, blk), lambda i, j: (i, 0, j)),
            pl.BlockSpec((1, 3, blk), lambda i, j: (i, 0, j)),
            pl.BlockSpec((1, 1, blk), lambda i, j: (i, 0, j)),
            pl.BlockSpec((9, FEAT), lambda i, j: (0, 0)),
            pl.BlockSpec((FEAT,), lambda i, j: (0,)),
        ],
        out_specs=pl.BlockSpec((1, FEAT, blk), lambda i, j: (i, 0, j)),
        out_shape=jax.ShapeDtypeStruct((NSETS, FEAT, NPAD), jnp.float32),
    )(pts, fcl, ids, W, b)


def _k26_body(pts_ref, fcl_ref, ids_ref, w_ref, b_ref, p0_ref):
    out = _featmul(pts_ref[...], fcl_ref[...], ids_ref[...], w_ref[...], b_ref[...])
    p0_ref[0] = out


def _k26(pts, fcl, ids, W, b):
    blk = 512
    return pl.pallas_call(
        _k26_body,
        grid=(2, NPAD // blk),
        in_specs=[
            pl.BlockSpec((1, 3, blk), lambda i, j: (i + 2, 0, j)),
            pl.BlockSpec((1, 3, blk), lambda i, j: (i + 2, 0, j)),
            pl.BlockSpec((1, 1, blk), lambda i, j: (i + 2, 0, j)),
            pl.BlockSpec((9, FEAT), lambda i, j: (0, 0)),
            pl.BlockSpec((FEAT,), lambda i, j: (0,)),
        ],
        out_specs=pl.BlockSpec((1, blk, FEAT), lambda i, j: (i, j, 0)),
        out_shape=jax.ShapeDtypeStruct((2, NPAD, FEAT), jnp.float32),
    )(pts, fcl, ids, W, b)


JOBS_PER_TILE = (NSETS * FEAT) // 32


def _k3_body(pft_hbm, ids_hbm, voxt_hbm, grid_v, idb, vb):
    c = lax.axis_index("c")
    s = lax.axis_index("s")
    wid = s * 2 + c
    zero16 = jnp.zeros((16,), jnp.float32)

    def zero_grid(i, _):
        off = pl.multiple_of(i * 16, 16)
        grid_v[pl.ds(off, 16)] = zero16
        return _

    for jl in range(JOBS_PER_TILE):
        job = wid * JOBS_PER_TILE + jl
        cb = job // FEAT
        f = job % FEAT
        lax.fori_loop(0, NCELL_PAD // 16, zero_grid, None)

        def run(ch, _):
            base = ch * CH
            pltpu.sync_copy(ids_hbm.at[pl.ds(cb * NPAD + base, CH)], idb)
            pltpu.sync_copy(
                pft_hbm.at[pl.ds((cb * FEAT + f) * NPAD + base, CH)], vb)

            def body(j, _2):
                off = pl.multiple_of(j * 16, 16)
                plsc.addupdate_scatter(grid_v, [idb[pl.ds(off, 16)]],
                                       vb[pl.ds(off, 16)])
                return _2

            lax.fori_loop(0, CH // 16, body, None)
            return _

        lax.fori_loop(0, NPAD // CH, run, None)
        pltpu.sync_copy(
            grid_v, voxt_hbm.at[pl.ds((cb * FEAT + f) * NCELL_PAD, NCELL_PAD)])


def _k3(pft, ids):
    return pl.kernel(
        _k3_body,
        out_type=jax.ShapeDtypeStruct((NSETS * FEAT * NCELL_PAD,), jnp.float32),
        mesh=_MESH,
        scratch_types=[
            pltpu.VMEM((NCELL_PAD,), jnp.float32),
            pltpu.VMEM((CH,), jnp.int32),
            pltpu.VMEM((CH,), jnp.float32),
        ],
        compiler_params=_SC_PARAMS,
    )(pft, ids)


def _k4_body(v1_ref, v0_ref, vh_ref, i1_ref, i0_ref, ih_ref, feat_ref, v1o_ref):
    v1 = v1_ref[0] * i1_ref[0, 0][None, :]
    v0 = v0_ref[0] * i0_ref[0, 0][None, :]
    vh = vh_ref[0] * ih_ref[0, 0][None, :]
    feat = v1 - 0.5 * (v0 + vh)
    feat_ref[0] = feat.T
    v1o_ref[0] = v1.T


def _k4(voxt, invd):
    blk = 512
    vspec = lambda off: pl.BlockSpec((1, FEAT, blk), lambda b, j: (b + off, 0, j))
    ispec = lambda off: pl.BlockSpec((1, 1, blk), lambda b, j: (b + off, 0, j))
    return pl.pallas_call(
        _k4_body,
        grid=(2, NUM_CELLS // blk),
        in_specs=[vspec(0), vspec(2), vspec(4), ispec(0), ispec(2), ispec(4)],
        out_specs=[
            pl.BlockSpec((1, blk, FEAT), lambda b, j: (b, j, 0)),
            pl.BlockSpec((1, blk, FEAT), lambda b, j: (b, j, 0)),
        ],
        out_shape=[
            jax.ShapeDtypeStruct((2, NUM_CELLS, FEAT), jnp.float32),
            jax.ShapeDtypeStruct((2, NUM_CELLS, FEAT), jnp.float32),
        ],
    )(voxt, voxt, voxt, invd, invd, invd)


def kernel(pc1s, pc0s, pch1s, W, b):
    pts = jnp.concatenate([pc1s, pc0s, pch1s], axis=0)
    pts = jnp.transpose(pts, (0, 2, 1))
    pts = jnp.pad(pts, ((0, 0), (0, 0), (0, NPAD - N_REAL)))

    ptsf = pts.reshape(-1)
    ids, gridat = _k1(ptsf)
    meant, invd = _k15(gridat.reshape(NSETS, 4, NCELL_PAD))
    fcl = _k2(ptsf, meant.reshape(-1))
    ids3 = ids.reshape(NSETS, 1, NPAD)
    fcl3 = fcl.reshape(NSETS, 3, NPAD)
    pft = _k25(pts, fcl3, ids3, W, b)
    p0f = _k26(pts, fcl3, ids3, W, b)
    voxt = _k3(pft.reshape(-1), ids)
    features, v1 = _k4(voxt.reshape(NSETS, FEAT, NCELL_PAD), invd)
    return features, v1, p0f[:, :N_REAL, :]


# speedup vs baseline: 2.0805x; 1.3195x over previous
import functools

import jax
import jax.numpy as jnp
from jax import lax
from jax.experimental import pallas as pl
from jax.experimental.pallas import tpu as pltpu
from jax.experimental.pallas import tpu_sc as plsc

GRID_X, GRID_Y = 256, 256
NUM_CELLS = GRID_X * GRID_Y
FEAT = 64
N_REAL = 100000
NPAD = 102400
PTS_PER_TILE = NPAD // 32
NCELL_PAD = 66048
TRASH = NUM_CELLS
NSETS = 6
CH = 2048

VX = 0.2
VY = 0.2
LOX = -25.6
LOY = -25.6

_MESH = plsc.VectorSubcoreMesh(core_axis_name="c", subcore_axis_name="s")
_SC_PARAMS = pltpu.CompilerParams(needs_layout_passes=False)


def _pillar_ids(xv, yv, gidx):
    cx = ((xv - LOX) / VX).astype(jnp.int32)
    cy = ((yv - LOY) / VY).astype(jnp.int32)
    cx = jnp.minimum(jnp.maximum(cx, 0), GRID_X - 1)
    cy = jnp.minimum(jnp.maximum(cy, 0), GRID_Y - 1)
    pid = cy * GRID_X + cx
    return jnp.where(gidx >= N_REAL, TRASH, pid)


def _k1_body(pts_hbm, ids_hbm, gridat_hbm, grid_v, xb, yb, sb, idb):
    c = lax.axis_index("c")
    s = lax.axis_index("s")
    wid = s * 2 + c
    cb = wid // 4
    col = wid % 4
    zero16 = jnp.zeros((16,), jnp.float32)
    iota = lax.iota(jnp.int32, 16)

    def zero_grid(i, _):
        off = pl.multiple_of(i * 16, 16)
        grid_v[pl.ds(off, 16)] = zero16
        return _

    def chunk_ids(ch, store_ids):
        base = ch * CH
        pltpu.sync_copy(pts_hbm.at[pl.ds((cb * 3 + 0) * NPAD + base, CH)], xb)
        pltpu.sync_copy(pts_hbm.at[pl.ds((cb * 3 + 1) * NPAD + base, CH)], yb)

        def body(j, _):
            off = pl.multiple_of(j * 16, 16)
            gidx = base + off + iota
            pid = _pillar_ids(xb[pl.ds(off, 16)], yb[pl.ds(off, 16)], gidx)
            if store_ids:
                idb[pl.ds(off, 16)] = pid
                plsc.addupdate_scatter(grid_v, [pid], jnp.ones((16,), jnp.float32))
            else:
                plsc.addupdate_scatter(grid_v, [pid], sb[pl.ds(off, 16)])
            return _

        return body

    @pl.when(jnp.logical_and(wid < 24, col == 0))
    def _():
        lax.fori_loop(0, NCELL_PAD // 16, zero_grid, None)

        def run(ch, _):
            lax.fori_loop(0, CH // 16, chunk_ids(ch, True), None)
            pltpu.sync_copy(idb, ids_hbm.at[pl.ds(cb * NPAD + ch * CH, CH)])
            return _

        lax.fori_loop(0, NPAD // CH, run, None)
        pltpu.sync_copy(grid_v, gridat_hbm.at[pl.ds(cb * 4 * NCELL_PAD, NCELL_PAD)])

    @pl.when(jnp.logical_and(wid < 24, col > 0))
    def _():
        lax.fori_loop(0, NCELL_PAD // 16, zero_grid, None)

        def run(ch, _):
            pltpu.sync_copy(
                pts_hbm.at[pl.ds((cb * 3 + col - 1) * NPAD + ch * CH, CH)], sb)
            lax.fori_loop(0, CH // 16, chunk_ids(ch, False), None)
            return _

        lax.fori_loop(0, NPAD // CH, run, None)
        pltpu.sync_copy(grid_v,
                        gridat_hbm.at[pl.ds((cb * 4 + col) * NCELL_PAD, NCELL_PAD)])


def _k1(pts):
    return pl.kernel(
        _k1_body,
        out_type=[
            jax.ShapeDtypeStruct((NSETS * NPAD,), jnp.int32),
            jax.ShapeDtypeStruct((NSETS * 4 * NCELL_PAD,), jnp.float32),
        ],
        mesh=_MESH,
        scratch_types=[
            pltpu.VMEM((NCELL_PAD,), jnp.float32),
            pltpu.VMEM((CH,), jnp.float32),
            pltpu.VMEM((CH,), jnp.float32),
            pltpu.VMEM((CH,), jnp.float32),
            pltpu.VMEM((CH,), jnp.int32),
        ],
        compiler_params=_SC_PARAMS,
    )(pts)


def _k15_body(ga_ref, mean_ref, invd_ref):
    cnt = ga_ref[0, 0, :]
    den = jnp.maximum(cnt, 1.0)
    mean_ref[0] = ga_ref[0, 1:4, :] / den[None, :]
    invd_ref[0, 0] = 1.0 / den


def _k15(gridat):
    return pl.pallas_call(
        _k15_body,
        grid=(NSETS,),
        in_specs=[pl.BlockSpec((1, 4, NCELL_PAD), lambda i: (i, 0, 0))],
        out_specs=[
            pl.BlockSpec((1, 3, NCELL_PAD), lambda i: (i, 0, 0)),
            pl.BlockSpec((1, 1, NCELL_PAD), lambda i: (i, 0, 0)),
        ],
        out_shape=[
            jax.ShapeDtypeStruct((NSETS, 3, NCELL_PAD), jnp.float32),
            jax.ShapeDtypeStruct((NSETS, 1, NCELL_PAD), jnp.float32),
        ],
    )(gridat)


def _k2_body(pts_hbm, mean_hbm, fcl_hbm, colb, xb, yb, zb, idb, fb):
    c = lax.axis_index("c")
    s = lax.axis_index("s")
    wid = s * 2 + c
    base = wid * PTS_PER_TILE
    iota = lax.iota(jnp.int32, 16)

    for cb in range(NSETS):
        pltpu.sync_copy(pts_hbm.at[pl.ds((cb * 3 + 0) * NPAD + base, PTS_PER_TILE)], xb)
        pltpu.sync_copy(pts_hbm.at[pl.ds((cb * 3 + 1) * NPAD + base, PTS_PER_TILE)], yb)
        pltpu.sync_copy(pts_hbm.at[pl.ds((cb * 3 + 2) * NPAD + base, PTS_PER_TILE)], zb)

        def mkids(j, _):
            off = pl.multiple_of(j * 16, 16)
            gidx = base + off + iota
            idb[pl.ds(off, 16)] = _pillar_ids(xb[pl.ds(off, 16)],
                                              yb[pl.ds(off, 16)], gidx)
            return _

        lax.fori_loop(0, PTS_PER_TILE // 16, mkids, None)

        for p, pbuf in enumerate((xb, yb, zb)):
            pltpu.sync_copy(
                mean_hbm.at[pl.ds((cb * 3 + p) * NCELL_PAD, NCELL_PAD)], colb)

            def gat(j, _, pbuf=pbuf):
                off = pl.multiple_of(j * 16, 16)
                pid = idb[pl.ds(off, 16)]
                m = plsc.load_gather(colb, [pid])
                fb[pl.ds(off, 16)] = pbuf[pl.ds(off, 16)] - m
                return _

            lax.fori_loop(0, PTS_PER_TILE // 16, gat, None)
            pltpu.sync_copy(
                fb, fcl_hbm.at[pl.ds((cb * 3 + p) * NPAD + base, PTS_PER_TILE)])


def _k2(pts, meant):
    return pl.kernel(
        _k2_body,
        out_type=jax.ShapeDtypeStruct((NSETS * 3 * NPAD,), jnp.float32),
        mesh=_MESH,
        scratch_types=[
            pltpu.VMEM((NCELL_PAD,), jnp.float32),
            pltpu.VMEM((PTS_PER_TILE,), jnp.float32),
            pltpu.VMEM((PTS_PER_TILE,), jnp.float32),
            pltpu.VMEM((PTS_PER_TILE,), jnp.float32),
            pltpu.VMEM((PTS_PER_TILE,), jnp.int32),
            pltpu.VMEM((PTS_PER_TILE,), jnp.float32),
        ],
        compiler_params=_SC_PARAMS,
    )(pts, meant)


def _featmul(pts_blk, fcl_blk, ids_blk, w_blk, b_blk):
    fid = ids_blk[0, 0]
    cx = (fid & (GRID_X - 1)).astype(jnp.float32)
    cy = (fid >> 8).astype(jnp.float32)
    vcx = (cx + 0.5) * VX + LOX
    vcy = (cy + 0.5) * VY + LOY
    fcx = pts_blk[0, 0, :] - vcx
    fcy = pts_blk[0, 1, :] - vcy
    fcz = pts_blk[0, 2, :]
    fcs = jnp.stack([fcx, fcy, fcz], axis=0)
    lhs = jnp.concatenate([pts_blk[0], fcl_blk[0], fcs], axis=0)
    out = lax.dot_general(lhs, w_blk,
                          dimension_numbers=(((0,), (0,)), ((), ())),
                          preferred_element_type=jnp.float32)
    return jax.nn.relu(out + b_blk[None, :])


def _k25_body(pts_ref, fcl_ref, ids_ref, w_ref, b_ref, pft_ref):
    out = _featmul(pts_ref[...], fcl_ref[...], ids_ref[...], w_ref[...], b_ref[...])
    pft_ref[0] = out.T


def _k25(pts, fcl, ids, W, b):
    blk = 512
    return pl.pallas_call(
        _k25_body,
        grid=(NSETS, NPAD // blk),
        in_specs=[
            pl.BlockSpec((1, 3, blk), lambda i, j: (i, 0, j)),
            pl.BlockSpec((1, 3, blk), lambda i, j: (i, 0, j)),
            pl.BlockSpec((1, 1, blk), lambda i, j: (i, 0, j)),
            pl.BlockSpec((9, FEAT), lambda i, j: (0, 0)),
            pl.BlockSpec((FEAT,), lambda i, j: (0,)),
        ],
        out_specs=pl.BlockSpec((1, FEAT, blk), lambda i, j: (i, 0, j)),
        out_shape=jax.ShapeDtypeStruct((NSETS, FEAT, NPAD), jnp.float32),
    )(pts, fcl, ids, W, b)


def _k26_body(pts_ref, fcl_ref, ids_ref, w_ref, b_ref, p0_ref):
    out = _featmul(pts_ref[...], fcl_ref[...], ids_ref[...], w_ref[...], b_ref[...])
    p0_ref[0] = out


def _k26(pts, fcl, ids, W, b):
    blk = 512
    return pl.pallas_call(
        _k26_body,
        grid=(2, NPAD // blk),
        in_specs=[
            pl.BlockSpec((1, 3, blk), lambda i, j: (i + 2, 0, j)),
            pl.BlockSpec((1, 3, blk), lambda i, j: (i + 2, 0, j)),
            pl.BlockSpec((1, 1, blk), lambda i, j: (i + 2, 0, j)),
            pl.BlockSpec((9, FEAT), lambda i, j: (0, 0)),
            pl.BlockSpec((FEAT,), lambda i, j: (0,)),
        ],
        out_specs=pl.BlockSpec((1, blk, FEAT), lambda i, j: (i, j, 0)),
        out_shape=jax.ShapeDtypeStruct((2, NPAD, FEAT), jnp.float32),
    )(pts, fcl, ids, W, b)


JOBS_PER_TILE = (NSETS * FEAT) // 32
CH3 = 3200
NCH3 = NPAD // CH3


def _k3_body(pft_hbm, ids_hbm, voxt_hbm, grid_v, ida, idb2, va, vb2, sa, sb, sf):
    c = lax.axis_index("c")
    s = lax.axis_index("s")
    wid = s * 2 + c
    zero16 = jnp.zeros((16,), jnp.float32)

    def zero_grid(i, _):
        off = pl.multiple_of(i * 16, 16)
        grid_v[pl.ds(off, 16)] = zero16
        return _

    def start(cb, f, ch, id_buf, v_buf, sem):
        base = ch * CH3
        pltpu.async_copy(ids_hbm.at[pl.ds(cb * NPAD + base, CH3)], id_buf, sem)
        pltpu.async_copy(
            pft_hbm.at[pl.ds((cb * FEAT + f) * NPAD + base, CH3)], v_buf, sem)

    def wait(cb, f, ch, id_buf, v_buf, sem):
        base = ch * CH3
        pltpu.make_async_copy(
            ids_hbm.at[pl.ds(cb * NPAD + base, CH3)], id_buf, sem).wait()
        pltpu.make_async_copy(
            pft_hbm.at[pl.ds((cb * FEAT + f) * NPAD + base, CH3)], v_buf, sem).wait()

    def scatter(id_buf, v_buf):
        def body(j, _):
            off = pl.multiple_of(j * 16, 16)
            plsc.addupdate_scatter(grid_v, [id_buf[pl.ds(off, 16)]],
                                   v_buf[pl.ds(off, 16)])
            return _
        lax.fori_loop(0, CH3 // 16, body, None)

    for jl in range(JOBS_PER_TILE):
        job = wid * JOBS_PER_TILE + jl
        cb = job // FEAT
        f = job % FEAT
        start(cb, f, 0, ida, va, sa)
        if jl > 0:
            pjob = wid * JOBS_PER_TILE + (jl - 1)
            pcb = pjob // FEAT
            pf = pjob % FEAT
            pltpu.make_async_copy(
                grid_v,
                voxt_hbm.at[pl.ds((pcb * FEAT + pf) * NCELL_PAD, NCELL_PAD)],
                sf).wait()
        lax.fori_loop(0, NCELL_PAD // 16, zero_grid, None)

        def outer(g, _):
            wait(cb, f, g, ida, va, sa)
            start(cb, f, g + 1, idb2, vb2, sb)
            scatter(ida, va)
            wait(cb, f, g + 1, idb2, vb2, sb)

            @pl.when(g + 2 < NCH3)
            def _():
                start(cb, f, g + 2, ida, va, sa)

            scatter(idb2, vb2)
            return _

        lax.fori_loop(0, NCH3 // 2, lambda i, _: outer(i * 2, _), None)
        pltpu.async_copy(
            grid_v, voxt_hbm.at[pl.ds((cb * FEAT + f) * NCELL_PAD, NCELL_PAD)], sf)

    ljob = wid * JOBS_PER_TILE + (JOBS_PER_TILE - 1)
    lcb = ljob // FEAT
    lf = ljob % FEAT
    pltpu.make_async_copy(
        grid_v, voxt_hbm.at[pl.ds((lcb * FEAT + lf) * NCELL_PAD, NCELL_PAD)],
        sf).wait()


def _k3(pft, ids):
    return pl.kernel(
        _k3_body,
        out_type=jax.ShapeDtypeStruct((NSETS * FEAT * NCELL_PAD,), jnp.float32),
        mesh=_MESH,
        scratch_types=[
            pltpu.VMEM((NCELL_PAD,), jnp.float32),
            pltpu.VMEM((CH3,), jnp.int32),
            pltpu.VMEM((CH3,), jnp.int32),
            pltpu.VMEM((CH3,), jnp.float32),
            pltpu.VMEM((CH3,), jnp.float32),
            pltpu.SemaphoreType.DMA,
            pltpu.SemaphoreType.DMA,
            pltpu.SemaphoreType.DMA,
        ],
        compiler_params=_SC_PARAMS,
    )(pft, ids)


def _k4_body(v1_ref, v0_ref, vh_ref, i1_ref, i0_ref, ih_ref, feat_ref, v1o_ref):
    v1 = v1_ref[0] * i1_ref[0, 0][None, :]
    v0 = v0_ref[0] * i0_ref[0, 0][None, :]
    vh = vh_ref[0] * ih_ref[0, 0][None, :]
    feat = v1 - 0.5 * (v0 + vh)
    feat_ref[0] = feat.T
    v1o_ref[0] = v1.T


def _k4(voxt, invd):
    blk = 512
    vspec = lambda off: pl.BlockSpec((1, FEAT, blk), lambda b, j: (b + off, 0, j))
    ispec = lambda off: pl.BlockSpec((1, 1, blk), lambda b, j: (b + off, 0, j))
    return pl.pallas_call(
        _k4_body,
        grid=(2, NUM_CELLS // blk),
        in_specs=[vspec(0), vspec(2), vspec(4), ispec(0), ispec(2), ispec(4)],
        out_specs=[
            pl.BlockSpec((1, blk, FEAT), lambda b, j: (b, j, 0)),
            pl.BlockSpec((1, blk, FEAT), lambda b, j: (b, j, 0)),
        ],
        out_shape=[
            jax.ShapeDtypeStruct((2, NUM_CELLS, FEAT), jnp.float32),
            jax.ShapeDtypeStruct((2, NUM_CELLS, FEAT), jnp.float32),
        ],
    )(voxt, voxt, voxt, invd, invd, invd)


def kernel(pc1s, pc0s, pch1s, W, b):
    pts = jnp.concatenate([pc1s, pc0s, pch1s], axis=0)
    pts = jnp.transpose(pts, (0, 2, 1))
    pts = jnp.pad(pts, ((0, 0), (0, 0), (0, NPAD - N_REAL)))

    ptsf = pts.reshape(-1)
    ids, gridat = _k1(ptsf)
    meant, invd = _k15(gridat.reshape(NSETS, 4, NCELL_PAD))
    fcl = _k2(ptsf, meant.reshape(-1))
    ids3 = ids.reshape(NSETS, 1, NPAD)
    fcl3 = fcl.reshape(NSETS, 3, NPAD)
    pft = _k25(pts, fcl3, ids3, W, b)
    p0f = _k26(pts, fcl3, ids3, W, b)
    voxt = _k3(pft.reshape(-1), ids)
    features, v1 = _k4(voxt.reshape(NSETS, FEAT, NCELL_PAD), invd)
    return features, v1, p0f[:, :N_REAL, :]


# speedup vs baseline: 3.2497x; 1.5620x over previous
import functools

import jax
import jax.numpy as jnp
from jax import lax
from jax.experimental import pallas as pl
from jax.experimental.pallas import tpu as pltpu
from jax.experimental.pallas import tpu_sc as plsc

GRID_X, GRID_Y = 256, 256
NUM_CELLS = GRID_X * GRID_Y
FEAT = 64
N_REAL = 100000
NPAD = 102400
PTS_PER_TILE = NPAD // 32
NCELL_PAD = 66048
TRASH = NUM_CELLS
NSETS = 6
CH = 2048

VX = 0.2
VY = 0.2
LOX = -25.6
LOY = -25.6

_MESH = plsc.VectorSubcoreMesh(core_axis_name="c", subcore_axis_name="s")
_SC_PARAMS = pltpu.CompilerParams(needs_layout_passes=False)


def _pillar_ids(xv, yv, gidx):
    cx = ((xv - LOX) / VX).astype(jnp.int32)
    cy = ((yv - LOY) / VY).astype(jnp.int32)
    cx = jnp.minimum(jnp.maximum(cx, 0), GRID_X - 1)
    cy = jnp.minimum(jnp.maximum(cy, 0), GRID_Y - 1)
    pid = cy * GRID_X + cx
    return jnp.where(gidx >= N_REAL, TRASH, pid)


NCH1 = NPAD // CH


def _k1_body(pts_hbm, ids_hbm, gridat_hbm, grid_v,
             xa, xb2, ya, yb2, sra, srb, ida, idb2, sia, sib, so):
    c = lax.axis_index("c")
    s = lax.axis_index("s")
    wid = s * 2 + c
    cb = wid // 4
    col = wid % 4
    iscnt = col == 0
    zero16 = jnp.zeros((16,), jnp.float32)
    ones16 = jnp.ones((16,), jnp.float32)
    iota = lax.iota(jnp.int32, 16)

    def zero_grid(i, _):
        base = pl.multiple_of(i * 128, 16)
        for u in range(8):
            grid_v[pl.ds(base + u * 16, 16)] = zero16
        return _

    def start_in(ch, xbuf, ybuf, sbuf, sem):
        base = ch * CH
        pltpu.async_copy(pts_hbm.at[pl.ds((cb * 3 + 0) * NPAD + base, CH)],
                         xbuf, sem)
        pltpu.async_copy(pts_hbm.at[pl.ds((cb * 3 + 1) * NPAD + base, CH)],
                         ybuf, sem)

        @pl.when(col > 0)
        def _():
            pltpu.async_copy(
                pts_hbm.at[pl.ds((cb * 3 + col - 1) * NPAD + base, CH)],
                sbuf, sem)

    def wait_in(ch, xbuf, ybuf, sbuf, sem):
        base = ch * CH
        pltpu.make_async_copy(
            pts_hbm.at[pl.ds((cb * 3 + 0) * NPAD + base, CH)], xbuf, sem).wait()
        pltpu.make_async_copy(
            pts_hbm.at[pl.ds((cb * 3 + 1) * NPAD + base, CH)], ybuf, sem).wait()

        @pl.when(col > 0)
        def _():
            pltpu.make_async_copy(
                pts_hbm.at[pl.ds((cb * 3 + col - 1) * NPAD + base, CH)],
                sbuf, sem).wait()

    def drain_out(idbuf):
        pltpu.make_async_copy(idbuf, ids_hbm.at[pl.ds(cb * NPAD, CH)], so).wait()

    def process(ch, xbuf, ybuf, sbuf, idbuf):
        base = ch * CH

        @pl.when(iscnt)
        def _():
            def bodyc(j, _):
                boff = pl.multiple_of(j * 64, 16)
                for u in range(4):
                    off = boff + u * 16
                    gidx = base + off + iota
                    pid = _pillar_ids(xbuf[pl.ds(off, 16)],
                                      ybuf[pl.ds(off, 16)], gidx)
                    idbuf[pl.ds(off, 16)] = pid
                    plsc.addupdate_scatter(grid_v, [pid], ones16)
                return _
            lax.fori_loop(0, CH // 64, bodyc, None)

        @pl.when(col > 0)
        def _():
            def bodys(j, _):
                boff = pl.multiple_of(j * 64, 16)
                for u in range(4):
                    off = boff + u * 16
                    gidx = base + off + iota
                    pid = _pillar_ids(xbuf[pl.ds(off, 16)],
                                      ybuf[pl.ds(off, 16)], gidx)
                    plsc.addupdate_scatter(grid_v, [pid], sbuf[pl.ds(off, 16)])
                return _
            lax.fori_loop(0, CH // 64, bodys, None)

    @pl.when(wid < 24)
    def _():
        start_in(0, xa, ya, sra, sia)
        lax.fori_loop(0, NCELL_PAD // 128, zero_grid, None)

        def outer(g, _):
            wait_in(g, xa, ya, sra, sia)
            start_in(g + 1, xb2, yb2, srb, sib)

            @pl.when(jnp.logical_and(iscnt, g >= 2))
            def _():
                drain_out(ida)

            process(g, xa, ya, sra, ida)

            @pl.when(iscnt)
            def _():
                pltpu.async_copy(ida, ids_hbm.at[pl.ds(cb * NPAD + g * CH, CH)],
                                 so)

            wait_in(g + 1, xb2, yb2, srb, sib)

            @pl.when(g + 2 < NCH1)
            def _():
                start_in(g + 2, xa, ya, sra, sia)

            @pl.when(jnp.logical_and(iscnt, g >= 2))
            def _():
                drain_out(idb2)

            process(g + 1, xb2, yb2, srb, idb2)

            @pl.when(iscnt)
            def _():
                pltpu.async_copy(
                    idb2, ids_hbm.at[pl.ds(cb * NPAD + (g + 1) * CH, CH)], so)

            return _

        lax.fori_loop(0, NCH1 // 2, lambda i, _: outer(i * 2, _), None)

        @pl.when(iscnt)
        def _():
            drain_out(ida)
            drain_out(idb2)

        pltpu.sync_copy(grid_v,
                        gridat_hbm.at[pl.ds((cb * 4 + col) * NCELL_PAD, NCELL_PAD)])


def _k1(pts):
    return pl.kernel(
        _k1_body,
        out_type=[
            jax.ShapeDtypeStruct((NSETS * NPAD,), jnp.int32),
            jax.ShapeDtypeStruct((NSETS * 4 * NCELL_PAD,), jnp.float32),
        ],
        mesh=_MESH,
        scratch_types=[
            pltpu.VMEM((NCELL_PAD,), jnp.float32),
            pltpu.VMEM((CH,), jnp.float32),
            pltpu.VMEM((CH,), jnp.float32),
            pltpu.VMEM((CH,), jnp.float32),
            pltpu.VMEM((CH,), jnp.float32),
            pltpu.VMEM((CH,), jnp.float32),
            pltpu.VMEM((CH,), jnp.float32),
            pltpu.VMEM((CH,), jnp.int32),
            pltpu.VMEM((CH,), jnp.int32),
            pltpu.SemaphoreType.DMA,
            pltpu.SemaphoreType.DMA,
            pltpu.SemaphoreType.DMA,
        ],
        compiler_params=_SC_PARAMS,
    )(pts)


def _k15_body(ga_ref, mean_ref, invd_ref):
    cnt = ga_ref[0, 0, :]
    den = jnp.maximum(cnt, 1.0)
    mean_ref[0] = ga_ref[0, 1:4, :] / den[None, :]
    invd_ref[0, 0] = 1.0 / den


def _k15(gridat):
    return pl.pallas_call(
        _k15_body,
        grid=(NSETS,),
        in_specs=[pl.BlockSpec((1, 4, NCELL_PAD), lambda i: (i, 0, 0))],
        out_specs=[
            pl.BlockSpec((1, 3, NCELL_PAD), lambda i: (i, 0, 0)),
            pl.BlockSpec((1, 1, NCELL_PAD), lambda i: (i, 0, 0)),
        ],
        out_shape=[
            jax.ShapeDtypeStruct((NSETS, 3, NCELL_PAD), jnp.float32),
            jax.ShapeDtypeStruct((NSETS, 1, NCELL_PAD), jnp.float32),
        ],
    )(gridat)


def _k2_body(pts_hbm, mean_hbm, fcl_hbm, colb, xb, yb, zb, idb, fb):
    c = lax.axis_index("c")
    s = lax.axis_index("s")
    wid = s * 2 + c
    base = wid * PTS_PER_TILE
    iota = lax.iota(jnp.int32, 16)

    for cb in range(NSETS):
        pltpu.sync_copy(pts_hbm.at[pl.ds((cb * 3 + 0) * NPAD + base, PTS_PER_TILE)], xb)
        pltpu.sync_copy(pts_hbm.at[pl.ds((cb * 3 + 1) * NPAD + base, PTS_PER_TILE)], yb)
        pltpu.sync_copy(pts_hbm.at[pl.ds((cb * 3 + 2) * NPAD + base, PTS_PER_TILE)], zb)

        def mkids(j, _):
            off = pl.multiple_of(j * 16, 16)
            gidx = base + off + iota
            idb[pl.ds(off, 16)] = _pillar_ids(xb[pl.ds(off, 16)],
                                              yb[pl.ds(off, 16)], gidx)
            return _

        lax.fori_loop(0, PTS_PER_TILE // 16, mkids, None)

        for p, pbuf in enumerate((xb, yb, zb)):
            pltpu.sync_copy(
                mean_hbm.at[pl.ds((cb * 3 + p) * NCELL_PAD, NCELL_PAD)], colb)

            def gat(j, _, pbuf=pbuf):
                off = pl.multiple_of(j * 16, 16)
                pid = idb[pl.ds(off, 16)]
                m = plsc.load_gather(colb, [pid])
                fb[pl.ds(off, 16)] = pbuf[pl.ds(off, 16)] - m
                return _

            lax.fori_loop(0, PTS_PER_TILE // 16, gat, None)
            pltpu.sync_copy(
                fb, fcl_hbm.at[pl.ds((cb * 3 + p) * NPAD + base, PTS_PER_TILE)])


def _k2(pts, meant):
    return pl.kernel(
        _k2_body,
        out_type=jax.ShapeDtypeStruct((NSETS * 3 * NPAD,), jnp.float32),
        mesh=_MESH,
        scratch_types=[
            pltpu.VMEM((NCELL_PAD,), jnp.float32),
            pltpu.VMEM((PTS_PER_TILE,), jnp.float32),
            pltpu.VMEM((PTS_PER_TILE,), jnp.float32),
            pltpu.VMEM((PTS_PER_TILE,), jnp.float32),
            pltpu.VMEM((PTS_PER_TILE,), jnp.int32),
            pltpu.VMEM((PTS_PER_TILE,), jnp.float32),
        ],
        compiler_params=_SC_PARAMS,
    )(pts, meant)


def _feat_lhs(pts_blk, fcl_blk, ids_blk):
    fid = ids_blk[0, 0]
    cx = (fid & (GRID_X - 1)).astype(jnp.float32)
    cy = (fid >> 8).astype(jnp.float32)
    vcx = (cx + 0.5) * VX + LOX
    vcy = (cy + 0.5) * VY + LOY
    fcx = pts_blk[0, 0, :] - vcx
    fcy = pts_blk[0, 1, :] - vcy
    fcz = pts_blk[0, 2, :]
    fcs = jnp.stack([fcx, fcy, fcz], axis=0)
    return jnp.concatenate([pts_blk[0], fcl_blk[0], fcs], axis=0)


def _k25_body(pts_ref, fcl_ref, ids_ref, w_ref, b_ref, pft_ref):
    lhs = _feat_lhs(pts_ref[...], fcl_ref[...], ids_ref[...])
    out = lax.dot_general(w_ref[...], lhs,
                          dimension_numbers=(((0,), (0,)), ((), ())),
                          preferred_element_type=jnp.float32)
    pft_ref[0] = jax.nn.relu(out + b_ref[...][:, None])


def _k25(pts, fcl, ids, W, b):
    blk = 2048
    return pl.pallas_call(
        _k25_body,
        grid=(NSETS, NPAD // blk),
        in_specs=[
            pl.BlockSpec((1, 3, blk), lambda i, j: (i, 0, j)),
            pl.BlockSpec((1, 3, blk), lambda i, j: (i, 0, j)),
            pl.BlockSpec((1, 1, blk), lambda i, j: (i, 0, j)),
            pl.BlockSpec((9, FEAT), lambda i, j: (0, 0)),
            pl.BlockSpec((FEAT,), lambda i, j: (0,)),
        ],
        out_specs=pl.BlockSpec((1, FEAT, blk), lambda i, j: (i, 0, j)),
        out_shape=jax.ShapeDtypeStruct((NSETS, FEAT, NPAD), jnp.float32),
    )(pts, fcl, ids, W, b)


def _k26_body(pts_ref, fcl_ref, ids_ref, w_ref, b_ref, p0_ref):
    lhs = _feat_lhs(pts_ref[...], fcl_ref[...], ids_ref[...])
    out = lax.dot_general(lhs, w_ref[...],
                          dimension_numbers=(((0,), (0,)), ((), ())),
                          preferred_element_type=jnp.float32)
    p0_ref[0] = jax.nn.relu(out + b_ref[...][None, :])


def _k26(pts, fcl, ids, W, b):
    blk = 2048
    return pl.pallas_call(
        _k26_body,
        grid=(2, NPAD // blk),
        in_specs=[
            pl.BlockSpec((1, 3, blk), lambda i, j: (i + 2, 0, j)),
            pl.BlockSpec((1, 3, blk), lambda i, j: (i + 2, 0, j)),
            pl.BlockSpec((1, 1, blk), lambda i, j: (i + 2, 0, j)),
            pl.BlockSpec((9, FEAT), lambda i, j: (0, 0)),
            pl.BlockSpec((FEAT,), lambda i, j: (0,)),
        ],
        out_specs=pl.BlockSpec((1, blk, FEAT), lambda i, j: (i, j, 0)),
        out_shape=jax.ShapeDtypeStruct((2, NPAD, FEAT), jnp.float32),
    )(pts, fcl, ids, W, b)


JOBS_PER_TILE = (NSETS * FEAT) // 32
CH3 = 3200
NCH3 = NPAD // CH3


def _k3_body(pft_hbm, ids_hbm, voxt_hbm, grid_v, ida, idb2, va, vb2, sa, sb, sf):
    c = lax.axis_index("c")
    s = lax.axis_index("s")
    wid = s * 2 + c
    zero16 = jnp.zeros((16,), jnp.float32)

    def zero_grid(i, _):
        base = pl.multiple_of(i * 128, 16)
        for u in range(8):
            grid_v[pl.ds(base + u * 16, 16)] = zero16
        return _

    def start(cb, f, ch, id_buf, v_buf, sem):
        base = ch * CH3
        pltpu.async_copy(ids_hbm.at[pl.ds(cb * NPAD + base, CH3)], id_buf, sem)
        pltpu.async_copy(
            pft_hbm.at[pl.ds((cb * FEAT + f) * NPAD + base, CH3)], v_buf, sem)

    def wait(cb, f, ch, id_buf, v_buf, sem):
        base = ch * CH3
        pltpu.make_async_copy(
            ids_hbm.at[pl.ds(cb * NPAD + base, CH3)], id_buf, sem).wait()
        pltpu.make_async_copy(
            pft_hbm.at[pl.ds((cb * FEAT + f) * NPAD + base, CH3)], v_buf, sem).wait()

    def scatter(id_buf, v_buf):
        U = 8
        def body(j, _):
            base = pl.multiple_of(j * (16 * U), 16)
            for u in range(U):
                off = base + u * 16
                plsc.addupdate_scatter(grid_v, [id_buf[pl.ds(off, 16)]],
                                       v_buf[pl.ds(off, 16)])
            return _
        lax.fori_loop(0, CH3 // (16 * U), body, None)

    for jl in range(JOBS_PER_TILE):
        job = wid * JOBS_PER_TILE + jl
        cb = job // FEAT
        f = job % FEAT
        start(cb, f, 0, ida, va, sa)
        if jl > 0:
            pjob = wid * JOBS_PER_TILE + (jl - 1)
            pcb = pjob // FEAT
            pf = pjob % FEAT
            pltpu.make_async_copy(
                grid_v,
                voxt_hbm.at[pl.ds((pcb * FEAT + pf) * NCELL_PAD, NCELL_PAD)],
                sf).wait()
        lax.fori_loop(0, NCELL_PAD // 128, zero_grid, None)

        def outer(g, _):
            wait(cb, f, g, ida, va, sa)
            start(cb, f, g + 1, idb2, vb2, sb)
            scatter(ida, va)
            wait(cb, f, g + 1, idb2, vb2, sb)

            @pl.when(g + 2 < NCH3)
            def _():
                start(cb, f, g + 2, ida, va, sa)

            scatter(idb2, vb2)
            return _

        lax.fori_loop(0, NCH3 // 2, lambda i, _: outer(i * 2, _), None)
        pltpu.async_copy(
            grid_v, voxt_hbm.at[pl.ds((cb * FEAT + f) * NCELL_PAD, NCELL_PAD)], sf)

    ljob = wid * JOBS_PER_TILE + (JOBS_PER_TILE - 1)
    lcb = ljob // FEAT
    lf = ljob % FEAT
    pltpu.make_async_copy(
        grid_v, voxt_hbm.at[pl.ds((lcb * FEAT + lf) * NCELL_PAD, NCELL_PAD)],
        sf).wait()


def _k3(pft, ids):
    return pl.kernel(
        _k3_body,
        out_type=jax.ShapeDtypeStruct((NSETS * FEAT * NCELL_PAD,), jnp.float32),
        mesh=_MESH,
        scratch_types=[
            pltpu.VMEM((NCELL_PAD,), jnp.float32),
            pltpu.VMEM((CH3,), jnp.int32),
            pltpu.VMEM((CH3,), jnp.int32),
            pltpu.VMEM((CH3,), jnp.float32),
            pltpu.VMEM((CH3,), jnp.float32),
            pltpu.SemaphoreType.DMA,
            pltpu.SemaphoreType.DMA,
            pltpu.SemaphoreType.DMA,
        ],
        compiler_params=_SC_PARAMS,
    )(pft, ids)


def _k4_body(v1_ref, v0_ref, vh_ref, i1_ref, i0_ref, ih_ref, feat_ref, v1o_ref):
    v1 = v1_ref[0] * i1_ref[0, 0][None, :]
    v0 = v0_ref[0] * i0_ref[0, 0][None, :]
    vh = vh_ref[0] * ih_ref[0, 0][None, :]
    feat = v1 - 0.5 * (v0 + vh)
    feat_ref[0] = feat.T
    v1o_ref[0] = v1.T


def _k4(voxt, invd):
    blk = 1024
    vspec = lambda off: pl.BlockSpec((1, FEAT, blk), lambda b, j: (b + off, 0, j))
    ispec = lambda off: pl.BlockSpec((1, 1, blk), lambda b, j: (b + off, 0, j))
    return pl.pallas_call(
        _k4_body,
        grid=(2, NUM_CELLS // blk),
        in_specs=[vspec(0), vspec(2), vspec(4), ispec(0), ispec(2), ispec(4)],
        out_specs=[
            pl.BlockSpec((1, blk, FEAT), lambda b, j: (b, j, 0)),
            pl.BlockSpec((1, blk, FEAT), lambda b, j: (b, j, 0)),
        ],
        out_shape=[
            jax.ShapeDtypeStruct((2, NUM_CELLS, FEAT), jnp.float32),
            jax.ShapeDtypeStruct((2, NUM_CELLS, FEAT), jnp.float32),
        ],
    )(voxt, voxt, voxt, invd, invd, invd)


def kernel(pc1s, pc0s, pch1s, W, b):
    pts = jnp.concatenate([pc1s, pc0s, pch1s], axis=0)
    pts = jnp.transpose(pts, (0, 2, 1))
    pts = jnp.pad(pts, ((0, 0), (0, 0), (0, NPAD - N_REAL)))

    ptsf = pts.reshape(-1)
    ids, gridat = _k1(ptsf)
    meant, invd = _k15(gridat.reshape(NSETS, 4, NCELL_PAD))
    fcl = _k2(ptsf, meant.reshape(-1))
    ids3 = ids.reshape(NSETS, 1, NPAD)
    fcl3 = fcl.reshape(NSETS, 3, NPAD)
    pft = _k25(pts, fcl3, ids3, W, b)
    p0f = _k26(pts, fcl3, ids3, W, b)
    voxt = _k3(pft.reshape(-1), ids)
    features, v1 = _k4(voxt.reshape(NSETS, FEAT, NCELL_PAD), invd)
    return features, v1, p0f[:, :N_REAL, :]
